# Initial kernel scaffold; baseline (speedup 1.0000x reference)
import jax
import jax.numpy as jnp
from jax.experimental import pallas as pl


def kernel(X, C, node_h, edge_h, edge_idx, mask_i, mask_ij, W_q, b_q, W_t, b_t, W_w, b_w, W_x, b_x):
    raise NotImplementedError("write your pallas kernel here")


# speedup vs baseline: 40.0551x; 40.0551x over previous
import dataclasses
import functools

import numpy as np
import jax
import jax.numpy as jnp
from jax import lax
from jax.experimental import pallas as pl
from jax.experimental.pallas import tpu as pltpu
from jax.experimental.pallas import tpu_sc as plsc

_EPS = 1e-3
_SCALE = 10.0
_GAMMA = 3.0 + 2.0 * np.sqrt(2.0)
_CPI8 = np.float32(np.cos(np.pi / 8.0))
_SPI8 = np.float32(np.sin(np.pi / 8.0))

_L_CO = np.float32(1.2297)
_A_CACO = np.float32(np.float32(122.52) * np.float32(np.pi / 180.0))
_D_DIH = np.float32(np.float32(180.0) * np.float32(np.pi / 180.0))
_COS_A = np.float32(np.cos(_A_CACO))
_SIN_A = np.float32(np.sin(_A_CACO))
_COS_D = np.float32(np.cos(_D_DIH))
_SIN_D = np.float32(np.sin(_D_DIH))


def _normed3(v, eps):
    inv = jax.lax.rsqrt(v[0] * v[0] + v[1] * v[1] + v[2] * v[2] + eps)
    return [v[0] * inv, v[1] * inv, v[2] * inv]


def _cross3(a, b):
    return [a[1] * b[2] - a[2] * b[1],
            a[2] * b[0] - a[0] * b[2],
            a[0] * b[1] - a[1] * b[0]]


def _frames_from_comps(xn, xca, xc, cm):
    u1 = [xn[i] - xca[i] for i in range(3)]
    n1 = _normed3(u1, _EPS)
    u2 = _normed3([xc[i] - xca[i] for i in range(3)], _EPS)
    d = n1[0] * u2[0] + n1[1] * u2[1] + n1[2] * u2[2]
    v = [u2[i] - d * n1[i] for i in range(3)]
    n2 = _normed3(v, _EPS)
    n3 = _cross3(n1, n2)
    r = [n1[0], n2[0], n3[0],
         n1[1], n2[1], n3[1],
         n1[2], n2[2], n3[2]]
    r = [cm * x for x in r]
    t = [cm * xca[i] for i in range(3)]
    return r, t


def _frames_body(x_ref, c_ref, o_ref):
    cm = (c_ref[...] > 0).astype(jnp.float32)
    xn = [x_ref[i] for i in range(3)]
    xca = [x_ref[3 + i] for i in range(3)]
    xc = [x_ref[6 + i] for i in range(3)]
    r, t = _frames_from_comps(xn, xca, xc, cm)
    for i in range(9):
        o_ref[i] = r[i]
    for i in range(3):
        o_ref[9 + i] = t[i]


def _frames_call(xt3, c2):
    rows, cols = c2.shape
    return pl.pallas_call(
        _frames_body,
        out_shape=jax.ShapeDtypeStruct((12, rows, cols), jnp.float32),
    )(xt3, c2)


_SC_NC = 2
_SC_NS = 16
_SC_CHUNK = 1024


def _gather_frames(table, flat_idx):
    E = flat_idx.shape[0]
    nw = _SC_NC * _SC_NS
    b_per_w = E // nw
    bn = table.shape[0]
    tab_rows = bn * 12 // 128
    mesh = plsc.VectorSubcoreMesh(core_axis_name="c", subcore_axis_name="s")
    cp = pltpu.CompilerParams()
    if "needs_layout_passes" in pltpu.CompilerParams.__dataclass_fields__:
        cp = dataclasses.replace(cp, needs_layout_passes=False)

    @functools.partial(
        pl.kernel, mesh=mesh, compiler_params=cp,
        out_type=jax.ShapeDtypeStruct((12, E), table.dtype),
        scratch_types=[
            pltpu.VMEM((tab_rows, 128), jnp.float32),
            pltpu.VMEM((_SC_CHUNK,), jnp.int32),
            pltpu.VMEM((12, _SC_CHUNK), jnp.float32),
            pltpu.SemaphoreType.DMA,
        ],
    )
    def k(table_hbm, idx_hbm, out_hbm, tab_v, idx_v, out_v, sem):
        wid = lax.axis_index("s") * _SC_NC + lax.axis_index("c")
        base = wid * b_per_w
        pltpu.sync_copy(table_hbm, tab_v)

        @pl.loop(0, b_per_w, step=_SC_CHUNK)
        def _(off):
            pltpu.sync_copy(idx_hbm.at[pl.ds(base + off, _SC_CHUNK)], idx_v)

            @pl.loop(0, _SC_CHUNK, step=16)
            def _(j):
                iv = idx_v[pl.ds(j, 16)]
                w0 = iv * 12
                for c in range(12):
                    w = w0 + c
                    out_v[c, pl.ds(j, 16)] = plsc.load_gather(
                        tab_v, [w >> 7, w & 127])

            pltpu.sync_copy(out_v, out_hbm.at[:, pl.ds(base + off, _SC_CHUNK)])

    return k(table.reshape(tab_rows, 128), flat_idx)


_NBLK = 128


def _edge_body(eh_ref, g_ref, m_ref, w_ref, b_ref, o_ref):
    nblk, kk = m_ref.shape
    h = eh_ref[...]
    p = jnp.dot(h, w_ref[...], preferred_element_type=jnp.float32) + b_ref[...]
    p3 = p.reshape(nblk, kk, 8)

    q0, q1, q2, q3 = (p3[:, :, i] for i in range(4))
    t0, t1, t2 = (p3[:, :, 4 + i] for i in range(3))
    lg = p3[:, :, 7]

    s = q0 * q0 + q1 * q1 + q2 * q2 + q3 * q3 + _EPS
    inv = 1.0 / s
    xx = q1 * q1; yy = q2 * q2; zz = q3 * q3
    xy = q1 * q2; xz = q1 * q3; yz = q2 * q3
    wx = q0 * q1; wy = q0 * q2; wz = q0 * q3
    rq = [s - 2.0 * (yy + zz), 2.0 * (xy - wz), 2.0 * (xz + wy),
          2.0 * (xy + wz), s - 2.0 * (xx + zz), 2.0 * (yz - wx),
          2.0 * (xz - wy), 2.0 * (yz + wx), s - 2.0 * (xx + yy)]

    rj = [g_ref[i] for i in range(9)]
    tj = [g_ref[9 + i] for i in range(3)]

    mx = jnp.max(lg, axis=1, keepdims=True)
    e = jnp.exp(lg - mx)
    se = jnp.sum(e, axis=1, keepdims=True)
    em = e * m_ref[...]
    sm = jnp.sum(em, axis=1, keepdims=True)
    w = em / (sm + 1e-5 * se)
    wt = w * inv

    outs = []
    for a in range(3):
        for c in range(3):
            rp = (rj[3 * a + 0] * rq[0 + c]
                  + rj[3 * a + 1] * rq[3 + c]
                  + rj[3 * a + 2] * rq[6 + c])
            outs.append(jnp.sum(wt * rp, axis=1, keepdims=True))
    for a in range(3):
        tp = tj[a] + rj[3 * a] * t0 + rj[3 * a + 1] * t1 + rj[3 * a + 2] * t2
        outs.append(jnp.sum(w * tp, axis=1, keepdims=True))
    o_ref[...] = jnp.concatenate(outs, axis=1)


def _edge_call(eh_flat, gathered, mask2, w_cat, b_cat):
    bn = mask2.shape[0]
    kk = mask2.shape[1]
    grid = (bn // _NBLK,)
    return pl.pallas_call(
        _edge_body,
        grid=grid,
        in_specs=[
            pl.BlockSpec((_NBLK * kk, 128), lambda i: (i, 0)),
            pl.BlockSpec((12, _NBLK, kk), lambda i: (0, i, 0)),
            pl.BlockSpec((_NBLK, kk), lambda i: (i, 0)),
            pl.BlockSpec((128, 8), lambda i: (0, 0)),
            pl.BlockSpec((1, 8), lambda i: (0, 0)),
        ],
        out_specs=pl.BlockSpec((_NBLK, 12), lambda i: (i, 0)),
        out_shape=jax.ShapeDtypeStruct((bn, 12), jnp.float32),
    )(eh_flat, gathered, mask2, w_cat, b_cat)


def _project_so3_comps(m):
    def M(a, b):
        return m[3 * a + b]

    s = {}
    for a in range(3):
        for b in range(a, 3):
            s[(a, b)] = M(0, a) * M(0, b) + M(1, a) * M(1, b) + M(2, a) * M(2, b)

    one = jnp.ones_like(m[0])
    zero = jnp.zeros_like(m[0])
    v = {(a, b): (one if a == b else zero) for a in range(3) for b in range(3)}

    def jacobi(p, q, r):
        spp, sqq, spq = s[(p, p)], s[(q, q)], s[(p, q)]
        ch = 2.0 * (spp - sqq)
        sh = spq
        use = (_GAMMA * sh * sh) < (ch * ch)
        wgt = jax.lax.rsqrt(ch * ch + sh * sh + 1e-38)
        ch = jnp.where(use, wgt * ch, _CPI8)
        sh = jnp.where(use, wgt * sh, _SPI8)
        c = ch * ch - sh * sh
        sn = 2.0 * ch * sh
        cc, ss, cs = c * c, sn * sn, c * sn
        npp = cc * spp + 2.0 * cs * spq + ss * sqq
        nqq = ss * spp - 2.0 * cs * spq + cc * sqq
        npq = cs * (sqq - spp) + (cc - ss) * spq
        pr = tuple(sorted((p, r)))
        qr = tuple(sorted((q, r)))
        spr, sqr = s[pr], s[qr]
        s[pr] = c * spr + sn * sqr
        s[qr] = -sn * spr + c * sqr
        s[(p, p)], s[(q, q)], s[(p, q)] = npp, nqq, npq
        for i in range(3):
            vp, vq = v[(i, p)], v[(i, q)]
            v[(i, p)] = c * vp + sn * vq
            v[(i, q)] = -sn * vp + c * vq

    for _ in range(4):
        jacobi(0, 1, 2)
        jacobi(0, 2, 1)
        jacobi(1, 2, 0)

    b = {}
    for i in range(3):
        for j in range(3):
            b[(i, j)] = M(i, 0) * v[(0, j)] + M(i, 1) * v[(1, j)] + M(i, 2) * v[(2, j)]

    def colnorm(j):
        return b[(0, j)] ** 2 + b[(1, j)] ** 2 + b[(2, j)] ** 2

    def condswap(p, q):
        do = colnorm(p) < colnorm(q)
        for i in range(3):
            bp, bq = b[(i, p)], b[(i, q)]
            b[(i, p)] = jnp.where(do, bq, bp)
            b[(i, q)] = jnp.where(do, -bp, bq)
            vp, vq = v[(i, p)], v[(i, q)]
            v[(i, p)] = jnp.where(do, vq, vp)
            v[(i, q)] = jnp.where(do, -vp, vq)

    condswap(0, 1)
    condswap(0, 2)
    condswap(1, 2)

    u = {(a, c): (one if a == c else zero) for a in range(3) for c in range(3)}

    def qr_givens(i, j):
        a1 = b[(i, i)]
        a2 = b[(j, i)]
        rho = jnp.sqrt(a1 * a1 + a2 * a2 + 1e-38)
        sh = jnp.where(rho > 1e-12, a2, zero)
        ch = jnp.abs(a1) + jnp.maximum(rho, 1e-12)
        neg = a1 < 0
        ch2 = jnp.where(neg, sh, ch)
        sh2 = jnp.where(neg, ch, sh)
        wgt = jax.lax.rsqrt(ch2 * ch2 + sh2 * sh2)
        ch2 = ch2 * wgt
        sh2 = sh2 * wgt
        c = ch2 * ch2 - sh2 * sh2
        sn = 2.0 * ch2 * sh2
        for col in range(3):
            bi, bj = b[(i, col)], b[(j, col)]
            b[(i, col)] = c * bi + sn * bj
            b[(j, col)] = -sn * bi + c * bj
        for rr in range(3):
            ui, uj = u[(rr, i)], u[(rr, j)]
            u[(rr, i)] = c * ui + sn * uj
            u[(rr, j)] = -sn * ui + c * uj

    qr_givens(0, 1)
    qr_givens(0, 2)
    qr_givens(1, 2)

    out = []
    for a in range(3):
        for c in range(3):
            out.append(u[(a, 0)] * v[(c, 0)] + u[(a, 1)] * v[(c, 1)]
                       + u[(a, 2)] * v[(c, 2)])
    return out


def _shift_next(x):
    fix = jnp.concatenate([x[1:, 0:1], x[0:1, 0:1]], axis=0)
    return jnp.concatenate([x[:, 1:], fix], axis=1)


def _residue_body(mt_ref, nh_ref, wx_ref, bx_ref, c_ref, mi_ref, o_ref, n_per_batch):
    rows, cols = c_ref.shape
    cm = (c_ref[...] > 0).astype(jnp.float32)
    mi = mi_ref[...]

    m = [mt_ref[i] for i in range(9)]
    ta = [mt_ref[9 + i] for i in range(3)]

    ra = _project_so3_comps(m)

    xn = [np.float32(1.459) * ra[3 * i] + ta[i] for i in range(3)]
    xca = list(ta)
    xc = [np.float32(-0.547) * ra[3 * i] + np.float32(-1.424) * ra[3 * i + 2] + ta[i]
          for i in range(3)]

    ri = jax.lax.broadcasted_iota(jnp.int32, (rows, cols), 0)
    li = jax.lax.broadcasted_iota(jnp.int32, (rows, cols), 1)
    flat = ri * cols + li
    not_last = ((flat % n_per_batch) != (n_per_batch - 1)).astype(jnp.float32)
    cnext = _shift_next(cm) * not_last
    xnn = [_shift_next(xn[i]) * cnext for i in range(3)]

    bc = _normed3([xc[i] - xca[i] for i in range(3)], _EPS)
    nn = _normed3(_cross3([xca[i] - xnn[i] for i in range(3)], bc), _EPS)
    mm = _cross3(nn, bc)
    k_bc = np.float32(-_L_CO * _COS_A)
    k_m = np.float32(_L_CO * _SIN_A * _COS_D)
    k_n = np.float32(_L_CO * _SIN_A * _SIN_D)
    xo = [xc[i] + k_bc * bc[i] + k_m * mm[i] + k_n * nn[i] for i in range(3)]

    xnew = [[cm * xn[i] for i in range(3)],
            [cm * xca[i] for i in range(3)],
            [cm * xc[i] for i in range(3)],
            [cm * xo[i] for i in range(3)]]

    r2, _ = _frames_from_comps(xnew[0], xnew[1], xnew[2], cm)
    dx = jnp.dot(nh_ref[...], wx_ref[...], preferred_element_type=jnp.float32) \
        + bx_ref[...]
    dxc = [dx[:, j:j + 1].reshape(rows, cols) for j in range(12)]
    msk = cm * mi
    for a in range(4):
        for i in range(3):
            g = (r2[3 * i + 0] * dxc[3 * a + 0]
                 + r2[3 * i + 1] * dxc[3 * a + 1]
                 + r2[3 * i + 2] * dxc[3 * a + 2])
            o_ref[3 * a + i] = xnew[a][i] + msk * g


def _residue_call(mt3, nh_flat, w_x, b_x, c2, mi2, n_per_batch):
    rows, cols = c2.shape

    def body(mt_ref, nh_ref, wx_ref, bx_ref, c_ref, mi_ref, o_ref):
        _residue_body(mt_ref, nh_ref, wx_ref, bx_ref, c_ref, mi_ref, o_ref,
                      n_per_batch)

    return pl.pallas_call(
        body,
        out_shape=jax.ShapeDtypeStruct((12, rows, cols), jnp.float32),
    )(mt3, nh_flat, w_x, b_x.reshape(1, 12), c2, mi2)


def kernel(X, C, node_h, edge_h, edge_idx, mask_i, mask_ij,
           W_q, b_q, W_t, b_t, W_w, b_w, W_x, b_x):
    B, N = C.shape
    K = edge_idx.shape[-1]
    BN = B * N
    cols = 128
    rows = BN // cols

    w_cat = jnp.concatenate([W_q, _SCALE * W_t, W_w], axis=1)
    b_cat = jnp.concatenate([b_q, _SCALE * b_t, b_w]).reshape(1, 8)

    xt3 = X.reshape(BN, 12).T.reshape(12, rows, cols)
    c2 = C.reshape(rows, cols)
    mi2 = mask_i.reshape(rows, cols)

    table = _frames_call(xt3, c2)
    table = table.reshape(12, BN).T

    flat_idx = (edge_idx + (jnp.arange(B, dtype=jnp.int32) * N)[:, None, None])
    flat_idx = flat_idx.reshape(BN * K).astype(jnp.int32)
    gathered = _gather_frames(table, flat_idx)
    gathered = gathered.reshape(12, BN, K)

    mt = _edge_call(edge_h.reshape(BN * K, 128), gathered,
                    mask_ij.reshape(BN, K), w_cat, b_cat)
    mt3 = mt.T.reshape(12, rows, cols)

    out = _residue_call(mt3, node_h.reshape(BN, 128), W_x, b_x, c2, mi2, N)
    return out.reshape(12, BN).T.reshape(B, N, 4, 3)


# speedup vs baseline: 445.6554x; 11.1261x over previous
import dataclasses
import functools

import numpy as np
import jax
import jax.numpy as jnp
from jax import lax
from jax.experimental import pallas as pl
from jax.experimental.pallas import tpu as pltpu
from jax.experimental.pallas import tpu_sc as plsc

_EPS = 1e-3
_SCALE = 10.0
_GAMMA = 3.0 + 2.0 * np.sqrt(2.0)
_CPI8 = np.float32(np.cos(np.pi / 8.0))
_SPI8 = np.float32(np.sin(np.pi / 8.0))

_L_CO = np.float32(1.2297)
_A_CACO = np.float32(np.float32(122.52) * np.float32(np.pi / 180.0))
_D_DIH = np.float32(np.float32(180.0) * np.float32(np.pi / 180.0))
_COS_A = np.float32(np.cos(_A_CACO))
_SIN_A = np.float32(np.sin(_A_CACO))
_COS_D = np.float32(np.cos(_D_DIH))
_SIN_D = np.float32(np.sin(_D_DIH))


def _normed3(v, eps):
    inv = jax.lax.rsqrt(v[0] * v[0] + v[1] * v[1] + v[2] * v[2] + eps)
    return [v[0] * inv, v[1] * inv, v[2] * inv]


def _cross3(a, b):
    return [a[1] * b[2] - a[2] * b[1],
            a[2] * b[0] - a[0] * b[2],
            a[0] * b[1] - a[1] * b[0]]


def _frames_from_comps(xn, xca, xc, cm):
    u1 = [xn[i] - xca[i] for i in range(3)]
    n1 = _normed3(u1, _EPS)
    u2 = _normed3([xc[i] - xca[i] for i in range(3)], _EPS)
    d = n1[0] * u2[0] + n1[1] * u2[1] + n1[2] * u2[2]
    v = [u2[i] - d * n1[i] for i in range(3)]
    n2 = _normed3(v, _EPS)
    n3 = _cross3(n1, n2)
    r = [n1[0], n2[0], n3[0],
         n1[1], n2[1], n3[1],
         n1[2], n2[2], n3[2]]
    r = [cm * x for x in r]
    t = [cm * xca[i] for i in range(3)]
    return r, t


def _frames_body(x_ref, c_ref, o_ref):
    cm = (c_ref[...] > 0).astype(jnp.float32)
    xn = [x_ref[i] for i in range(3)]
    xca = [x_ref[3 + i] for i in range(3)]
    xc = [x_ref[6 + i] for i in range(3)]
    r, t = _frames_from_comps(xn, xca, xc, cm)
    for i in range(9):
        o_ref[i] = r[i]
    for i in range(3):
        o_ref[9 + i] = t[i]


def _frames_call(xt3, c2):
    rows, cols = c2.shape
    return pl.pallas_call(
        _frames_body,
        out_shape=jax.ShapeDtypeStruct((12, rows, cols), jnp.float32),
    )(xt3, c2)


_SC_NC = 2
_SC_NS = 16
_SC_CHUNK = 1024


def _gather_frames(table, flat_idx):
    E = flat_idx.shape[0]
    nw = _SC_NC * _SC_NS
    b_per_w = E // nw
    bn = table.shape[0]
    tab_rows = bn * 12 // 128
    mesh = plsc.VectorSubcoreMesh(core_axis_name="c", subcore_axis_name="s")
    cp = pltpu.CompilerParams()
    if "needs_layout_passes" in pltpu.CompilerParams.__dataclass_fields__:
        cp = dataclasses.replace(cp, needs_layout_passes=False)

    @functools.partial(
        pl.kernel, mesh=mesh, compiler_params=cp,
        out_type=jax.ShapeDtypeStruct((12, E), table.dtype),
        scratch_types=[
            pltpu.VMEM((tab_rows, 128), jnp.float32),
            pltpu.VMEM((_SC_CHUNK,), jnp.int32),
            pltpu.VMEM((12, _SC_CHUNK), jnp.float32),
            pltpu.SemaphoreType.DMA,
        ],
    )
    def k(table_hbm, idx_hbm, out_hbm, tab_v, idx_v, out_v, sem):
        wid = lax.axis_index("s") * _SC_NC + lax.axis_index("c")
        base = wid * b_per_w
        pltpu.sync_copy(table_hbm, tab_v)

        @pl.loop(0, b_per_w, step=_SC_CHUNK)
        def _(off):
            pltpu.sync_copy(idx_hbm.at[pl.ds(base + off, _SC_CHUNK)], idx_v)

            @pl.loop(0, _SC_CHUNK, step=16)
            def _(j):
                iv = idx_v[pl.ds(j, 16)]
                w0 = iv * 12
                for c in range(12):
                    w = w0 + c
                    out_v[c, pl.ds(j, 16)] = plsc.load_gather(
                        tab_v, [w >> 7, w & 127])

            pltpu.sync_copy(out_v, out_hbm.at[:, pl.ds(base + off, _SC_CHUNK)])

    return k(table.reshape(tab_rows, 128), flat_idx)


_NBLK = 128


def _eye_f32(n):
    ri = jax.lax.broadcasted_iota(jnp.int32, (n, n), 0)
    ci = jax.lax.broadcasted_iota(jnp.int32, (n, n), 1)
    return (ri == ci).astype(jnp.float32)


def _edge_body(eh_ref, g_ref, m_ref, w_ref, b_ref, s_ref, o_ref):
    h = eh_ref[...]
    p = jnp.dot(h, w_ref[...], preferred_element_type=jnp.float32)
    pt = jax.lax.dot_general(_eye_f32(8), p, (((1,), (1,)), ((), ())),
                             preferred_element_type=jnp.float32)
    pt = pt + b_ref[...]

    q0, q1, q2, q3 = (pt[i:i + 1, :] for i in range(4))
    t0, t1, t2 = (pt[4 + i:5 + i, :] for i in range(3))
    lg = pt[7:8, :]

    s = q0 * q0 + q1 * q1 + q2 * q2 + q3 * q3 + _EPS
    inv = 1.0 / s
    xx = q1 * q1; yy = q2 * q2; zz = q3 * q3
    xy = q1 * q2; xz = q1 * q3; yz = q2 * q3
    wx = q0 * q1; wy = q0 * q2; wz = q0 * q3
    rq = [s - 2.0 * (yy + zz), 2.0 * (xy - wz), 2.0 * (xz + wy),
          2.0 * (xy + wz), s - 2.0 * (xx + zz), 2.0 * (yz - wx),
          2.0 * (xz - wy), 2.0 * (yz + wx), s - 2.0 * (xx + yy)]

    rj = [g_ref[i:i + 1, :] for i in range(9)]
    tj = [g_ref[9 + i:10 + i, :] for i in range(3)]

    e = jnp.exp(lg)
    em = e * m_ref[...]
    ew = em * inv

    rows = []
    for a in range(3):
        for c in range(3):
            rp = (rj[3 * a + 0] * rq[0 + c]
                  + rj[3 * a + 1] * rq[3 + c]
                  + rj[3 * a + 2] * rq[6 + c])
            rows.append(ew * rp)
    for a in range(3):
        tp = tj[a] + rj[3 * a] * t0 + rj[3 * a + 1] * t1 + rj[3 * a + 2] * t2
        rows.append(em * tp)
    rows.append(em)
    rows.append(e)
    y = jnp.concatenate(rows, axis=0)

    red = jnp.dot(y, s_ref[...], preferred_element_type=jnp.float32)
    den = 1.0 / (red[12:13, :] + 1e-5 * red[13:14, :])
    o_ref[...] = red[0:12, :] * den


def _edge_call(eh_flat, gathered, mask_flat, w_cat, b_cat, smat):
    e_total = mask_flat.shape[1]
    bn = e_total // (smat.shape[0] // _NBLK)
    kk = e_total // bn
    eblk = _NBLK * kk
    grid = (bn // _NBLK,)
    return pl.pallas_call(
        _edge_body,
        grid=grid,
        in_specs=[
            pl.BlockSpec((eblk, 128), lambda i: (i, 0)),
            pl.BlockSpec((12, eblk), lambda i: (0, i)),
            pl.BlockSpec((1, eblk), lambda i: (0, i)),
            pl.BlockSpec((128, 8), lambda i: (0, 0)),
            pl.BlockSpec((8, 1), lambda i: (0, 0)),
            pl.BlockSpec((eblk, _NBLK), lambda i: (0, 0)),
        ],
        out_specs=pl.BlockSpec((12, _NBLK), lambda i: (0, i)),
        out_shape=jax.ShapeDtypeStruct((12, bn), jnp.float32),
    )(eh_flat, gathered, mask_flat, w_cat, b_cat, smat)


def _project_so3_comps(m):
    def M(a, b):
        return m[3 * a + b]

    s = {}
    for a in range(3):
        for b in range(a, 3):
            s[(a, b)] = M(0, a) * M(0, b) + M(1, a) * M(1, b) + M(2, a) * M(2, b)

    one = jnp.ones_like(m[0])
    zero = jnp.zeros_like(m[0])
    v = {(a, b): (one if a == b else zero) for a in range(3) for b in range(3)}

    def jacobi(p, q, r):
        spp, sqq, spq = s[(p, p)], s[(q, q)], s[(p, q)]
        ch = 2.0 * (spp - sqq)
        sh = spq
        use = (_GAMMA * sh * sh) < (ch * ch)
        wgt = jax.lax.rsqrt(ch * ch + sh * sh + 1e-38)
        ch = jnp.where(use, wgt * ch, _CPI8)
        sh = jnp.where(use, wgt * sh, _SPI8)
        c = ch * ch - sh * sh
        sn = 2.0 * ch * sh
        cc, ss, cs = c * c, sn * sn, c * sn
        npp = cc * spp + 2.0 * cs * spq + ss * sqq
        nqq = ss * spp - 2.0 * cs * spq + cc * sqq
        npq = cs * (sqq - spp) + (cc - ss) * spq
        pr = tuple(sorted((p, r)))
        qr = tuple(sorted((q, r)))
        spr, sqr = s[pr], s[qr]
        s[pr] = c * spr + sn * sqr
        s[qr] = -sn * spr + c * sqr
        s[(p, p)], s[(q, q)], s[(p, q)] = npp, nqq, npq
        for i in range(3):
            vp, vq = v[(i, p)], v[(i, q)]
            v[(i, p)] = c * vp + sn * vq
            v[(i, q)] = -sn * vp + c * vq

    for _ in range(4):
        jacobi(0, 1, 2)
        jacobi(0, 2, 1)
        jacobi(1, 2, 0)

    b = {}
    for i in range(3):
        for j in range(3):
            b[(i, j)] = M(i, 0) * v[(0, j)] + M(i, 1) * v[(1, j)] + M(i, 2) * v[(2, j)]

    def colnorm(j):
        return b[(0, j)] ** 2 + b[(1, j)] ** 2 + b[(2, j)] ** 2

    def condswap(p, q):
        do = colnorm(p) < colnorm(q)
        for i in range(3):
            bp, bq = b[(i, p)], b[(i, q)]
            b[(i, p)] = jnp.where(do, bq, bp)
            b[(i, q)] = jnp.where(do, -bp, bq)
            vp, vq = v[(i, p)], v[(i, q)]
            v[(i, p)] = jnp.where(do, vq, vp)
            v[(i, q)] = jnp.where(do, -vp, vq)

    condswap(0, 1)
    condswap(0, 2)
    condswap(1, 2)

    u = {(a, c): (one if a == c else zero) for a in range(3) for c in range(3)}

    def qr_givens(i, j):
        a1 = b[(i, i)]
        a2 = b[(j, i)]
        rho = jnp.sqrt(a1 * a1 + a2 * a2 + 1e-38)
        sh = jnp.where(rho > 1e-12, a2, zero)
        ch = jnp.abs(a1) + jnp.maximum(rho, 1e-12)
        neg = a1 < 0
        ch2 = jnp.where(neg, sh, ch)
        sh2 = jnp.where(neg, ch, sh)
        wgt = jax.lax.rsqrt(ch2 * ch2 + sh2 * sh2)
        ch2 = ch2 * wgt
        sh2 = sh2 * wgt
        c = ch2 * ch2 - sh2 * sh2
        sn = 2.0 * ch2 * sh2
        for col in range(3):
            bi, bj = b[(i, col)], b[(j, col)]
            b[(i, col)] = c * bi + sn * bj
            b[(j, col)] = -sn * bi + c * bj
        for rr in range(3):
            ui, uj = u[(rr, i)], u[(rr, j)]
            u[(rr, i)] = c * ui + sn * uj
            u[(rr, j)] = -sn * ui + c * uj

    qr_givens(0, 1)
    qr_givens(0, 2)
    qr_givens(1, 2)

    out = []
    for a in range(3):
        for c in range(3):
            out.append(u[(a, 0)] * v[(c, 0)] + u[(a, 1)] * v[(c, 1)]
                       + u[(a, 2)] * v[(c, 2)])
    return out


def _shift_next(x):
    fix = jnp.concatenate([x[1:, 0:1], x[0:1, 0:1]], axis=0)
    return jnp.concatenate([x[:, 1:], fix], axis=1)


def _residue_body(mt_ref, nh_ref, wx_ref, bx_ref, c_ref, mi_ref, o_ref, n_per_batch):
    rows, cols = c_ref.shape
    cm = (c_ref[...] > 0).astype(jnp.float32)
    mi = mi_ref[...]

    m = [mt_ref[i] for i in range(9)]
    ta = [mt_ref[9 + i] for i in range(3)]

    ra = _project_so3_comps(m)

    xn = [np.float32(1.459) * ra[3 * i] + ta[i] for i in range(3)]
    xca = list(ta)
    xc = [np.float32(-0.547) * ra[3 * i] + np.float32(-1.424) * ra[3 * i + 2] + ta[i]
          for i in range(3)]

    ri = jax.lax.broadcasted_iota(jnp.int32, (rows, cols), 0)
    li = jax.lax.broadcasted_iota(jnp.int32, (rows, cols), 1)
    flat = ri * cols + li
    not_last = ((flat % n_per_batch) != (n_per_batch - 1)).astype(jnp.float32)
    cnext = _shift_next(cm) * not_last
    xnn = [_shift_next(xn[i]) * cnext for i in range(3)]

    bc = _normed3([xc[i] - xca[i] for i in range(3)], _EPS)
    nn = _normed3(_cross3([xca[i] - xnn[i] for i in range(3)], bc), _EPS)
    mm = _cross3(nn, bc)
    k_bc = np.float32(-_L_CO * _COS_A)
    k_m = np.float32(_L_CO * _SIN_A * _COS_D)
    k_n = np.float32(_L_CO * _SIN_A * _SIN_D)
    xo = [xc[i] + k_bc * bc[i] + k_m * mm[i] + k_n * nn[i] for i in range(3)]

    xnew = [[cm * xn[i] for i in range(3)],
            [cm * xca[i] for i in range(3)],
            [cm * xc[i] for i in range(3)],
            [cm * xo[i] for i in range(3)]]

    r2, _ = _frames_from_comps(xnew[0], xnew[1], xnew[2], cm)
    dx = jnp.dot(nh_ref[...], wx_ref[...], preferred_element_type=jnp.float32)
    dxt = jax.lax.dot_general(_eye_f32(12), dx, (((1,), (1,)), ((), ())),
                              preferred_element_type=jnp.float32)
    dxt = (dxt + bx_ref[...]).reshape(12, rows, cols)
    dxc = [dxt[j] for j in range(12)]
    msk = cm * mi
    for a in range(4):
        for i in range(3):
            g = (r2[3 * i + 0] * dxc[3 * a + 0]
                 + r2[3 * i + 1] * dxc[3 * a + 1]
                 + r2[3 * i + 2] * dxc[3 * a + 2])
            o_ref[3 * a + i] = xnew[a][i] + msk * g


def _residue_call(mt3, nh_flat, w_x, b_x, c2, mi2, n_per_batch):
    rows, cols = c2.shape

    def body(mt_ref, nh_ref, wx_ref, bx_ref, c_ref, mi_ref, o_ref):
        _residue_body(mt_ref, nh_ref, wx_ref, bx_ref, c_ref, mi_ref, o_ref,
                      n_per_batch)

    return pl.pallas_call(
        body,
        out_shape=jax.ShapeDtypeStruct((12, rows, cols), jnp.float32),
    )(mt3, nh_flat, w_x, b_x.reshape(12, 1), c2, mi2)


def kernel(X, C, node_h, edge_h, edge_idx, mask_i, mask_ij,
           W_q, b_q, W_t, b_t, W_w, b_w, W_x, b_x):
    B, N = C.shape
    K = edge_idx.shape[-1]
    BN = B * N
    cols = 128
    rows = BN // cols

    w_cat = jnp.concatenate([W_q, _SCALE * W_t, W_w], axis=1)
    b_cat = jnp.concatenate([b_q, _SCALE * b_t, b_w]).reshape(8, 1)
    smat = jnp.repeat(jnp.eye(cols, dtype=jnp.float32), K, axis=0)

    xt3 = X.reshape(BN, 12).T.reshape(12, rows, cols)
    c2 = C.reshape(rows, cols)
    mi2 = mask_i.reshape(rows, cols)

    table = _frames_call(xt3, c2)
    table = table.reshape(12, BN).T

    flat_idx = (edge_idx + (jnp.arange(B, dtype=jnp.int32) * N)[:, None, None])
    flat_idx = flat_idx.reshape(BN * K).astype(jnp.int32)
    gathered = _gather_frames(table, flat_idx)

    mt = _edge_call(edge_h.reshape(BN * K, 128), gathered,
                    mask_ij.reshape(1, BN * K), w_cat, b_cat, smat)
    mt3 = mt.reshape(12, rows, cols)

    out = _residue_call(mt3, node_h.reshape(BN, 128), W_x, b_x, c2, mi2, N)
    return out.reshape(12, BN).T.reshape(B, N, 4, 3)


# speedup vs baseline: 501.0140x; 1.1242x over previous
import dataclasses
import functools

import numpy as np
import jax
import jax.numpy as jnp
from jax import lax
from jax.experimental import pallas as pl
from jax.experimental.pallas import tpu as pltpu
from jax.experimental.pallas import tpu_sc as plsc

_EPS = 1e-3
_SCALE = 10.0
_GAMMA = 3.0 + 2.0 * np.sqrt(2.0)
_CPI8 = np.float32(np.cos(np.pi / 8.0))
_SPI8 = np.float32(np.sin(np.pi / 8.0))

_L_CO = np.float32(1.2297)
_A_CACO = np.float32(np.float32(122.52) * np.float32(np.pi / 180.0))
_D_DIH = np.float32(np.float32(180.0) * np.float32(np.pi / 180.0))
_COS_A = np.float32(np.cos(_A_CACO))
_SIN_A = np.float32(np.sin(_A_CACO))
_COS_D = np.float32(np.cos(_D_DIH))
_SIN_D = np.float32(np.sin(_D_DIH))


def _normed3(v, eps):
    inv = jax.lax.rsqrt(v[0] * v[0] + v[1] * v[1] + v[2] * v[2] + eps)
    return [v[0] * inv, v[1] * inv, v[2] * inv]


def _cross3(a, b):
    return [a[1] * b[2] - a[2] * b[1],
            a[2] * b[0] - a[0] * b[2],
            a[0] * b[1] - a[1] * b[0]]


def _frames_from_comps(xn, xca, xc, cm):
    u1 = [xn[i] - xca[i] for i in range(3)]
    n1 = _normed3(u1, _EPS)
    u2 = _normed3([xc[i] - xca[i] for i in range(3)], _EPS)
    d = n1[0] * u2[0] + n1[1] * u2[1] + n1[2] * u2[2]
    v = [u2[i] - d * n1[i] for i in range(3)]
    n2 = _normed3(v, _EPS)
    n3 = _cross3(n1, n2)
    r = [n1[0], n2[0], n3[0],
         n1[1], n2[1], n3[1],
         n1[2], n2[2], n3[2]]
    r = [cm * x for x in r]
    t = [cm * xca[i] for i in range(3)]
    return r, t


def _frames_body(x_ref, c_ref, o_ref):
    cm = (c_ref[...] > 0).astype(jnp.float32)
    xn = [x_ref[i] for i in range(3)]
    xca = [x_ref[3 + i] for i in range(3)]
    xc = [x_ref[6 + i] for i in range(3)]
    r, t = _frames_from_comps(xn, xca, xc, cm)
    for i in range(9):
        o_ref[i] = r[i]
    for i in range(3):
        o_ref[9 + i] = t[i]


def _frames_call(xt3, c2):
    rows, cols = c2.shape
    return pl.pallas_call(
        _frames_body,
        out_shape=jax.ShapeDtypeStruct((12, rows, cols), jnp.float32),
    )(xt3, c2)


_SC_NC = 2
_SC_NS = 16
_SC_CHUNK = 512


def _gather_frames(table, flat_idx, n_per_batch, kk):
    E = flat_idx.shape[0]
    nw = _SC_NC * _SC_NS
    b_per_w = E // nw
    n_chunks = b_per_w // _SC_CHUNK
    tab_rows = table.shape[0]
    bn = tab_rows * 128 // 12
    ek = n_per_batch * kk
    mesh = plsc.VectorSubcoreMesh(core_axis_name="c", subcore_axis_name="s")
    cp = pltpu.CompilerParams()
    if "needs_layout_passes" in pltpu.CompilerParams.__dataclass_fields__:
        cp = dataclasses.replace(cp, needs_layout_passes=False)

    @functools.partial(
        pl.kernel, mesh=mesh, compiler_params=cp,
        out_type=jax.ShapeDtypeStruct((12, E), table.dtype),
        scratch_types=[
            pltpu.VMEM((tab_rows, 128), jnp.float32),
            pltpu.VMEM((_SC_CHUNK,), jnp.int32),
            pltpu.VMEM((_SC_CHUNK,), jnp.int32),
            pltpu.VMEM((12, _SC_CHUNK), jnp.float32),
            pltpu.VMEM((12, _SC_CHUNK), jnp.float32),
            pltpu.SemaphoreType.DMA,
            pltpu.SemaphoreType.DMA,
            pltpu.SemaphoreType.DMA,
            pltpu.SemaphoreType.DMA,
        ],
    )
    def k(table_hbm, idx_hbm, out_hbm, tab_v, idx_v0, idx_v1, out_v0, out_v1,
          sem_i0, sem_i1, sem_o0, sem_o1):
        wid = lax.axis_index("s") * _SC_NC + lax.axis_index("c")
        base = wid * b_per_w
        ibufs = (idx_v0, idx_v1)
        obufs = (out_v0, out_v1)
        isems = (sem_i0, sem_i1)
        osems = (sem_o0, sem_o1)
        pltpu.sync_copy(table_hbm, tab_v)

        pltpu.async_copy(idx_hbm.at[pl.ds(base, _SC_CHUNK)], idx_v0, sem_i0)
        for g in range(n_chunks):
            bi = g % 2
            off = base + g * _SC_CHUNK
            pltpu.make_async_copy(
                idx_hbm.at[pl.ds(off, _SC_CHUNK)], ibufs[bi], isems[bi]).wait()
            if g + 1 < n_chunks:
                nxt = base + (g + 1) * _SC_CHUNK
                pltpu.async_copy(
                    idx_hbm.at[pl.ds(nxt, _SC_CHUNK)], ibufs[(g + 1) % 2],
                    isems[(g + 1) % 2])
            if g >= 2:
                prev = base + (g - 2) * _SC_CHUNK
                pltpu.make_async_copy(
                    obufs[bi], out_hbm.at[:, pl.ds(prev, _SC_CHUNK)],
                    osems[bi]).wait()
            boff = (off // ek) * n_per_batch
            ib = ibufs[bi]
            ob = obufs[bi]

            @pl.loop(0, _SC_CHUNK, step=16)
            def _(j):
                iv = ib[pl.ds(j, 16)] + boff
                hi = iv >> 7
                lo = iv & 127
                for c in range(12):
                    ob[c, pl.ds(j, 16)] = plsc.load_gather(
                        tab_v, [hi + (c * bn) // 128, lo])

            pltpu.async_copy(ob, out_hbm.at[:, pl.ds(off, _SC_CHUNK)],
                             osems[bi])
        for g in (n_chunks - 2, n_chunks - 1):
            bi = g % 2
            off = base + g * _SC_CHUNK
            pltpu.make_async_copy(
                obufs[bi], out_hbm.at[:, pl.ds(off, _SC_CHUNK)],
                osems[bi]).wait()

    return k(table, flat_idx)


_NBLK = 128


def _eye_f32(n):
    ri = jax.lax.broadcasted_iota(jnp.int32, (n, n), 0)
    ci = jax.lax.broadcasted_iota(jnp.int32, (n, n), 1)
    return (ri == ci).astype(jnp.float32)


def _edge_body(eh_ref, g_ref, m_ref, w_ref, b_ref, s_ref, o_ref):
    h = eh_ref[...]
    p = jnp.dot(h, w_ref[...], preferred_element_type=jnp.float32)
    pt = jax.lax.dot_general(_eye_f32(8), p, (((1,), (1,)), ((), ())),
                             preferred_element_type=jnp.float32)
    pt = pt + b_ref[...]

    q0, q1, q2, q3 = (pt[i:i + 1, :] for i in range(4))
    t0, t1, t2 = (pt[4 + i:5 + i, :] for i in range(3))
    lg = pt[7:8, :]

    s = q0 * q0 + q1 * q1 + q2 * q2 + q3 * q3 + _EPS
    inv = 1.0 / s
    xx = q1 * q1; yy = q2 * q2; zz = q3 * q3
    xy = q1 * q2; xz = q1 * q3; yz = q2 * q3
    wx = q0 * q1; wy = q0 * q2; wz = q0 * q3
    rq = [s - 2.0 * (yy + zz), 2.0 * (xy - wz), 2.0 * (xz + wy),
          2.0 * (xy + wz), s - 2.0 * (xx + zz), 2.0 * (yz - wx),
          2.0 * (xz - wy), 2.0 * (yz + wx), s - 2.0 * (xx + yy)]

    rj = [g_ref[i:i + 1, :] for i in range(9)]
    tj = [g_ref[9 + i:10 + i, :] for i in range(3)]

    e = jnp.exp(lg)
    em = e * m_ref[...]
    ew = em * inv

    rows = []
    for a in range(3):
        for c in range(3):
            rp = (rj[3 * a + 0] * rq[0 + c]
                  + rj[3 * a + 1] * rq[3 + c]
                  + rj[3 * a + 2] * rq[6 + c])
            rows.append(ew * rp)
    for a in range(3):
        tp = tj[a] + rj[3 * a] * t0 + rj[3 * a + 1] * t1 + rj[3 * a + 2] * t2
        rows.append(em * tp)
    rows.append(em)
    rows.append(e)
    y = jnp.concatenate(rows, axis=0)

    red = jnp.dot(y, s_ref[...], preferred_element_type=jnp.float32)
    den = 1.0 / (red[12:13, :] + 1e-5 * red[13:14, :])
    o_ref[...] = red[0:12, :] * den


def _edge_call(eh_flat, gathered, mask_flat, w_cat, b_cat, smat):
    e_total = mask_flat.shape[1]
    bn = e_total // (smat.shape[0] // _NBLK)
    kk = e_total // bn
    eblk = _NBLK * kk
    grid = (bn // _NBLK,)
    return pl.pallas_call(
        _edge_body,
        grid=grid,
        in_specs=[
            pl.BlockSpec((eblk, 128), lambda i: (i, 0)),
            pl.BlockSpec((12, eblk), lambda i: (0, i)),
            pl.BlockSpec((1, eblk), lambda i: (0, i)),
            pl.BlockSpec((128, 8), lambda i: (0, 0)),
            pl.BlockSpec((8, 1), lambda i: (0, 0)),
            pl.BlockSpec((eblk, _NBLK), lambda i: (0, 0)),
        ],
        out_specs=pl.BlockSpec((12, _NBLK), lambda i: (0, i)),
        out_shape=jax.ShapeDtypeStruct((12, bn), jnp.float32),
    )(eh_flat, gathered, mask_flat, w_cat, b_cat, smat)


def _project_so3_comps(m):
    def M(a, b):
        return m[3 * a + b]

    s = {}
    for a in range(3):
        for b in range(a, 3):
            s[(a, b)] = M(0, a) * M(0, b) + M(1, a) * M(1, b) + M(2, a) * M(2, b)

    one = jnp.ones_like(m[0])
    zero = jnp.zeros_like(m[0])
    v = {(a, b): (one if a == b else zero) for a in range(3) for b in range(3)}

    def jacobi(p, q, r):
        spp, sqq, spq = s[(p, p)], s[(q, q)], s[(p, q)]
        ch = 2.0 * (spp - sqq)
        sh = spq
        use = (_GAMMA * sh * sh) < (ch * ch)
        wgt = jax.lax.rsqrt(ch * ch + sh * sh + 1e-38)
        ch = jnp.where(use, wgt * ch, _CPI8)
        sh = jnp.where(use, wgt * sh, _SPI8)
        c = ch * ch - sh * sh
        sn = 2.0 * ch * sh
        cc, ss, cs = c * c, sn * sn, c * sn
        npp = cc * spp + 2.0 * cs * spq + ss * sqq
        nqq = ss * spp - 2.0 * cs * spq + cc * sqq
        npq = cs * (sqq - spp) + (cc - ss) * spq
        pr = tuple(sorted((p, r)))
        qr = tuple(sorted((q, r)))
        spr, sqr = s[pr], s[qr]
        s[pr] = c * spr + sn * sqr
        s[qr] = -sn * spr + c * sqr
        s[(p, p)], s[(q, q)], s[(p, q)] = npp, nqq, npq
        for i in range(3):
            vp, vq = v[(i, p)], v[(i, q)]
            v[(i, p)] = c * vp + sn * vq
            v[(i, q)] = -sn * vp + c * vq

    for _ in range(4):
        jacobi(0, 1, 2)
        jacobi(0, 2, 1)
        jacobi(1, 2, 0)

    b = {}
    for i in range(3):
        for j in range(3):
            b[(i, j)] = M(i, 0) * v[(0, j)] + M(i, 1) * v[(1, j)] + M(i, 2) * v[(2, j)]

    def colnorm(j):
        return b[(0, j)] ** 2 + b[(1, j)] ** 2 + b[(2, j)] ** 2

    def condswap(p, q):
        do = colnorm(p) < colnorm(q)
        for i in range(3):
            bp, bq = b[(i, p)], b[(i, q)]
            b[(i, p)] = jnp.where(do, bq, bp)
            b[(i, q)] = jnp.where(do, -bp, bq)
            vp, vq = v[(i, p)], v[(i, q)]
            v[(i, p)] = jnp.where(do, vq, vp)
            v[(i, q)] = jnp.where(do, -vp, vq)

    condswap(0, 1)
    condswap(0, 2)
    condswap(1, 2)

    u = {(a, c): (one if a == c else zero) for a in range(3) for c in range(3)}

    def qr_givens(i, j):
        a1 = b[(i, i)]
        a2 = b[(j, i)]
        rho = jnp.sqrt(a1 * a1 + a2 * a2 + 1e-38)
        sh = jnp.where(rho > 1e-12, a2, zero)
        ch = jnp.abs(a1) + jnp.maximum(rho, 1e-12)
        neg = a1 < 0
        ch2 = jnp.where(neg, sh, ch)
        sh2 = jnp.where(neg, ch, sh)
        wgt = jax.lax.rsqrt(ch2 * ch2 + sh2 * sh2)
        ch2 = ch2 * wgt
        sh2 = sh2 * wgt
        c = ch2 * ch2 - sh2 * sh2
        sn = 2.0 * ch2 * sh2
        for col in range(3):
            bi, bj = b[(i, col)], b[(j, col)]
            b[(i, col)] = c * bi + sn * bj
            b[(j, col)] = -sn * bi + c * bj
        for rr in range(3):
            ui, uj = u[(rr, i)], u[(rr, j)]
            u[(rr, i)] = c * ui + sn * uj
            u[(rr, j)] = -sn * ui + c * uj

    qr_givens(0, 1)
    qr_givens(0, 2)
    qr_givens(1, 2)

    out = []
    for a in range(3):
        for c in range(3):
            out.append(u[(a, 0)] * v[(c, 0)] + u[(a, 1)] * v[(c, 1)]
                       + u[(a, 2)] * v[(c, 2)])
    return out


def _shift_next(x):
    fix = jnp.concatenate([x[1:, 0:1], x[0:1, 0:1]], axis=0)
    return jnp.concatenate([x[:, 1:], fix], axis=1)


def _residue_body(mt_ref, nh_ref, wx_ref, bx_ref, c_ref, mi_ref, o_ref, n_per_batch):
    rows, cols = c_ref.shape
    cm = (c_ref[...] > 0).astype(jnp.float32)
    mi = mi_ref[...]

    m = [mt_ref[i] for i in range(9)]
    ta = [mt_ref[9 + i] for i in range(3)]

    ra = _project_so3_comps(m)

    xn = [np.float32(1.459) * ra[3 * i] + ta[i] for i in range(3)]
    xca = list(ta)
    xc = [np.float32(-0.547) * ra[3 * i] + np.float32(-1.424) * ra[3 * i + 2] + ta[i]
          for i in range(3)]

    ri = jax.lax.broadcasted_iota(jnp.int32, (rows, cols), 0)
    li = jax.lax.broadcasted_iota(jnp.int32, (rows, cols), 1)
    flat = ri * cols + li
    not_last = ((flat % n_per_batch) != (n_per_batch - 1)).astype(jnp.float32)
    cnext = _shift_next(cm) * not_last
    xnn = [_shift_next(xn[i]) * cnext for i in range(3)]

    bc = _normed3([xc[i] - xca[i] for i in range(3)], _EPS)
    nn = _normed3(_cross3([xca[i] - xnn[i] for i in range(3)], bc), _EPS)
    mm = _cross3(nn, bc)
    k_bc = np.float32(-_L_CO * _COS_A)
    k_m = np.float32(_L_CO * _SIN_A * _COS_D)
    k_n = np.float32(_L_CO * _SIN_A * _SIN_D)
    xo = [xc[i] + k_bc * bc[i] + k_m * mm[i] + k_n * nn[i] for i in range(3)]

    xnew = [[cm * xn[i] for i in range(3)],
            [cm * xca[i] for i in range(3)],
            [cm * xc[i] for i in range(3)],
            [cm * xo[i] for i in range(3)]]

    r2, _ = _frames_from_comps(xnew[0], xnew[1], xnew[2], cm)
    dx = jnp.dot(nh_ref[...], wx_ref[...], preferred_element_type=jnp.float32)
    dxt = jax.lax.dot_general(_eye_f32(12), dx, (((1,), (1,)), ((), ())),
                              preferred_element_type=jnp.float32)
    dxt = (dxt + bx_ref[...]).reshape(12, rows, cols)
    dxc = [dxt[j] for j in range(12)]
    msk = cm * mi
    for a in range(4):
        for i in range(3):
            g = (r2[3 * i + 0] * dxc[3 * a + 0]
                 + r2[3 * i + 1] * dxc[3 * a + 1]
                 + r2[3 * i + 2] * dxc[3 * a + 2])
            o_ref[3 * a + i] = xnew[a][i] + msk * g


def _residue_call(mt3, nh_flat, w_x, b_x, c2, mi2, n_per_batch):
    rows, cols = c2.shape

    def body(mt_ref, nh_ref, wx_ref, bx_ref, c_ref, mi_ref, o_ref):
        _residue_body(mt_ref, nh_ref, wx_ref, bx_ref, c_ref, mi_ref, o_ref,
                      n_per_batch)

    return pl.pallas_call(
        body,
        out_shape=jax.ShapeDtypeStruct((12, rows, cols), jnp.float32),
    )(mt3, nh_flat, w_x, b_x.reshape(12, 1), c2, mi2)


def kernel(X, C, node_h, edge_h, edge_idx, mask_i, mask_ij,
           W_q, b_q, W_t, b_t, W_w, b_w, W_x, b_x):
    B, N = C.shape
    K = edge_idx.shape[-1]
    BN = B * N
    cols = 128
    rows = BN // cols

    w_cat = jnp.concatenate([W_q, _SCALE * W_t, W_w], axis=1)
    b_cat = jnp.concatenate([b_q, _SCALE * b_t, b_w]).reshape(8, 1)
    smat = jnp.repeat(jnp.eye(cols, dtype=jnp.float32), K, axis=0)

    xt3 = X.reshape(BN, 12).T.reshape(12, rows, cols)
    c2 = C.reshape(rows, cols)
    mi2 = mask_i.reshape(rows, cols)

    table = _frames_call(xt3, c2)
    table = table.reshape(12 * BN // 128, 128)

    flat_idx = edge_idx.reshape(BN * K).astype(jnp.int32)
    gathered = _gather_frames(table, flat_idx, N, K)

    mt = _edge_call(edge_h.reshape(BN * K, 128), gathered,
                    mask_ij.reshape(1, BN * K), w_cat, b_cat, smat)
    mt3 = mt.reshape(12, rows, cols)

    out = _residue_call(mt3, node_h.reshape(BN, 128), W_x, b_x, c2, mi2, N)
    return out.reshape(12, BN).T.reshape(B, N, 4, 3)
